# baseline (speedup 1.0000x reference)
import functools

import jax
import jax.numpy as jnp
from jax import lax
from jax.experimental import pallas as pl
from jax.experimental.pallas import tpu as pltpu
from jax.experimental.pallas import tpu_sc as plsc

NUM_EMBEDDINGS = 1000000
TABLE_DIM = 56
SHARED_DIM = 8
OUT_DIM = TABLE_DIM + SHARED_DIM
BATCH = 16384

NC = 2
NS = 16
NW = NC * NS
B_PER_W = BATCH // NW
CHUNK = 128
N_CHUNKS = B_PER_W // CHUNK


@functools.partial(
    pl.kernel,
    out_type=jax.ShapeDtypeStruct((BATCH, OUT_DIM), jnp.float32),
    mesh=plsc.VectorSubcoreMesh(
        core_axis_name="c", subcore_axis_name="s", num_cores=NC,
        num_subcores=NS),
    scratch_types=[
        pltpu.VMEM((N_CHUNKS, CHUNK), jnp.int32),
        pltpu.VMEM((B_PER_W, TABLE_DIM), jnp.float32),
        pltpu.VMEM((B_PER_W, SHARED_DIM), jnp.float32),
        pltpu.VMEM((16,), jnp.float32),
        pltpu.SemaphoreType.DMA,
    ],
    compiler_params=pltpu.CompilerParams(
        use_tc_tiling_on_sc=False, needs_layout_passes=False),
)
def _sc_embed(x_hbm, table_hbm, shared_hbm, out_hbm, idx_v, rows_v, sh_v,
              sh16_v, sem):
    wid = lax.axis_index("s") * NC + lax.axis_index("c")
    base = wid * B_PER_W

    pltpu.sync_copy(x_hbm.at[pl.ds(wid * N_CHUNKS, N_CHUNKS)], idx_v)

    copies = []
    for j in range(N_CHUNKS):
        copies.append(pltpu.async_copy(
            table_hbm.at[idx_v.at[j]],
            rows_v.at[pl.ds(j * CHUNK, CHUNK)],
            sem,
        ))

    pltpu.sync_copy(shared_hbm, sh16_v.at[pl.ds(0, SHARED_DIM)])
    pltpu.sync_copy(shared_hbm, sh16_v.at[pl.ds(SHARED_DIM, SHARED_DIM)])
    val = sh16_v[...]
    iota = lax.iota(jnp.int32, 16)
    rowpat = lax.shift_right_logical(iota, 3)
    colpat = lax.bitwise_and(iota, 7)

    def fill_body(i, carry):
        rows = lax.add(lax.broadcast(i * 2, (16,)), rowpat)
        plsc.store_scatter(sh_v, [rows, colpat], val)
        return carry

    lax.fori_loop(0, B_PER_W // 2, fill_body, 0)

    for c in copies:
        c.wait()

    pltpu.sync_copy(rows_v, out_hbm.at[pl.ds(base, B_PER_W),
                                       pl.ds(0, TABLE_DIM)])
    pltpu.sync_copy(sh_v, out_hbm.at[pl.ds(base, B_PER_W),
                                     pl.ds(TABLE_DIM, SHARED_DIM)])


def kernel(x, table, shared_embed):
    x2 = x.astype(jnp.int32).reshape(BATCH // CHUNK, CHUNK)
    shared_flat = shared_embed.reshape(SHARED_DIM).astype(jnp.float32)
    out = _sc_embed(x2, table, shared_flat)
    return out[:, None, :]


# speedup vs baseline: 4.0030x; 4.0030x over previous
import functools

import jax
import jax.numpy as jnp
from jax import lax
from jax.experimental import pallas as pl
from jax.experimental.pallas import tpu as pltpu
from jax.experimental.pallas import tpu_sc as plsc

NUM_EMBEDDINGS = 1000000
TABLE_DIM = 56
SHARED_DIM = 8
OUT_DIM = TABLE_DIM + SHARED_DIM
BATCH = 16384

NC = 2
NS = 16
NW = NC * NS
B_PER_W = BATCH // NW
HALF = B_PER_W // 2


@functools.partial(
    pl.kernel,
    out_type=jax.ShapeDtypeStruct((BATCH, OUT_DIM), jnp.float32),
    mesh=plsc.VectorSubcoreMesh(
        core_axis_name="c", subcore_axis_name="s", num_cores=NC,
        num_subcores=NS),
    scratch_types=[
        pltpu.VMEM((B_PER_W,), jnp.int32),
        pltpu.VMEM((B_PER_W, TABLE_DIM), jnp.float32),
        pltpu.VMEM((HALF, OUT_DIM), jnp.float32),
        pltpu.VMEM((16,), jnp.float32),
        pltpu.SemaphoreType.DMA,
    ],
    compiler_params=pltpu.CompilerParams(needs_layout_passes=False),
)
def _sc_embed(x_hbm, table_hbm, shared_hbm, out_hbm, idx_v, rows_v, out_v,
              sh16_v, sem):
    wid = lax.axis_index("s") * NC + lax.axis_index("c")
    base = wid * B_PER_W

    pltpu.sync_copy(x_hbm.at[pl.ds(base, B_PER_W)], idx_v)

    def fire(g, carry):
        vec = idx_v[pl.ds(g * 16, 16)]
        for k in range(16):
            r = vec[k]
            pltpu.async_copy(
                table_hbm.at[pl.ds(r, 1), :],
                rows_v.at[pl.ds(g * 16 + k, 1), :],
                sem,
            )
        return carry

    lax.fori_loop(0, B_PER_W // 16, fire, 0)

    pltpu.sync_copy(shared_hbm, sh16_v.at[pl.ds(0, SHARED_DIM)])
    pltpu.sync_copy(shared_hbm, sh16_v.at[pl.ds(SHARED_DIM, SHARED_DIM)])
    sh = sh16_v[...]
    iota = lax.iota(jnp.int32, 16)
    rowpat = lax.shift_right_logical(iota, 3)
    shcols = lax.add(lax.bitwise_and(iota, 7),
                     lax.broadcast(TABLE_DIM, (16,)))
    tailcols = lax.add(iota, lax.broadcast(40, (16,)))
    tailmask = iota >= 8

    pltpu.make_async_copy(
        table_hbm.at[pl.ds(0, B_PER_W), :], rows_v, sem).wait()

    for h in range(2):
        def fill_body(i, carry):
            rows = lax.add(lax.broadcast(i * 2, (16,)), rowpat)
            plsc.store_scatter(out_v, [rows, shcols], sh)
            return carry

        lax.fori_loop(0, HALF // 2, fill_body, 0)

        def row_body(r, carry):
            src = h * HALF + r
            out_v[r, pl.ds(0, 16)] = rows_v[src, pl.ds(0, 16)]
            out_v[r, pl.ds(16, 16)] = rows_v[src, pl.ds(16, 16)]
            out_v[r, pl.ds(32, 16)] = rows_v[src, pl.ds(32, 16)]
            tail = rows_v[src, pl.ds(40, 16)]
            rvec = lax.broadcast(r, (16,))
            plsc.store_scatter(out_v, [rvec, tailcols], tail, mask=tailmask)
            return carry

        lax.fori_loop(0, HALF, row_body, 0)

        pltpu.sync_copy(out_v, out_hbm.at[pl.ds(base + h * HALF, HALF)])


def kernel(x, table, shared_embed):
    x1 = x.astype(jnp.int32)
    shared_flat = shared_embed.reshape(SHARED_DIM).astype(jnp.float32)
    out = _sc_embed(x1, table, shared_flat)
    return out[:, None, :]


# speedup vs baseline: 4.0439x; 1.0102x over previous
import functools

import jax
import jax.numpy as jnp
from jax import lax
from jax.experimental import pallas as pl
from jax.experimental.pallas import tpu as pltpu
from jax.experimental.pallas import tpu_sc as plsc

NUM_EMBEDDINGS = 1000000
TABLE_DIM = 56
SHARED_DIM = 8
OUT_DIM = TABLE_DIM + SHARED_DIM
BATCH = 16384

NC = 2
NS = 16
NW = NC * NS
B_PER_W = BATCH // NW
HALF = B_PER_W // 2


@functools.partial(
    pl.kernel,
    out_type=jax.ShapeDtypeStruct((BATCH, OUT_DIM), jnp.float32),
    mesh=plsc.VectorSubcoreMesh(
        core_axis_name="c", subcore_axis_name="s", num_cores=NC,
        num_subcores=NS),
    scratch_types=[
        pltpu.VMEM((B_PER_W,), jnp.int32),
        pltpu.VMEM((B_PER_W, TABLE_DIM), jnp.float32),
        pltpu.VMEM((HALF, OUT_DIM), jnp.float32),
        pltpu.VMEM((16,), jnp.float32),
        pltpu.SemaphoreType.DMA,
    ],
    compiler_params=pltpu.CompilerParams(needs_layout_passes=False),
)
def _sc_embed(x_hbm, table_hbm, shared_hbm, out_hbm, idx_v, rows_v, out_v,
              sh16_v, sem):
    wid = lax.axis_index("s") * NC + lax.axis_index("c")
    base = wid * B_PER_W

    pltpu.sync_copy(x_hbm.at[pl.ds(base, B_PER_W)], idx_v)

    @plsc.parallel_loop(0, B_PER_W // 16)
    def fire(g):
        vec = idx_v[pl.ds(g * 16, 16)]
        for k in range(16):
            r = vec[k]
            pltpu.async_copy(
                table_hbm.at[pl.ds(r, 1), :],
                rows_v.at[pl.ds(g * 16 + k, 1), :],
                sem,
            )

    pltpu.sync_copy(shared_hbm, sh16_v.at[pl.ds(0, SHARED_DIM)])
    pltpu.sync_copy(shared_hbm, sh16_v.at[pl.ds(SHARED_DIM, SHARED_DIM)])
    sh = sh16_v[...]
    iota = lax.iota(jnp.int32, 16)
    rowpat = lax.shift_right_logical(iota, 3)
    shcols = lax.add(lax.bitwise_and(iota, 7),
                     lax.broadcast(TABLE_DIM, (16,)))
    tailcols = lax.add(iota, lax.broadcast(40, (16,)))
    tailmask = iota >= 8

    pltpu.make_async_copy(
        table_hbm.at[pl.ds(0, B_PER_W), :], rows_v, sem).wait()

    for h in range(2):
        @plsc.parallel_loop(0, HALF // 2)
        def fill_body(i):
            rows = lax.add(lax.broadcast(i * 2, (16,)), rowpat)
            plsc.store_scatter(out_v, [rows, shcols], sh)

        @plsc.parallel_loop(0, HALF)
        def row_body(r):
            src = h * HALF + r
            out_v[r, pl.ds(0, 16)] = rows_v[src, pl.ds(0, 16)]
            out_v[r, pl.ds(16, 16)] = rows_v[src, pl.ds(16, 16)]
            out_v[r, pl.ds(32, 16)] = rows_v[src, pl.ds(32, 16)]
            tail = rows_v[src, pl.ds(40, 16)]
            rvec = lax.broadcast(r, (16,))
            plsc.store_scatter(out_v, [rvec, tailcols], tail, mask=tailmask)

        pltpu.sync_copy(out_v, out_hbm.at[pl.ds(base + h * HALF, HALF)])


def kernel(x, table, shared_embed):
    x1 = x.astype(jnp.int32)
    shared_flat = shared_embed.reshape(SHARED_DIM).astype(jnp.float32)
    out = _sc_embed(x1, table, shared_flat)
    return out[:, None, :]
